# initial kernel scaffold (unmeasured)
import jax
import jax.numpy as jnp
from jax import lax
from jax.experimental import pallas as pl
from jax.experimental.pallas import tpu as pltpu

N_DEV = 4
CAP = 158


def _a2a_pallas(payload):
    _, rows, d = payload.shape

    def body(payload_ref, recv_ref, local_sem, send_sems, recv_sems):
        me = lax.axis_index("i")

        barrier = pltpu.get_barrier_semaphore()
        for k in range(1, N_DEV):
            pl.semaphore_signal(
                barrier,
                inc=1,
                device_id=((me + k) % N_DEV,),
                device_id_type=pl.DeviceIdType.MESH,
            )
        pl.semaphore_wait(barrier, N_DEV - 1)

        local_copy = pltpu.make_async_copy(
            payload_ref.at[me], recv_ref.at[me], local_sem
        )
        local_copy.start()

        rdmas = []
        for k in range(1, N_DEV):
            p = (me + k) % N_DEV
            rdma = pltpu.make_async_remote_copy(
                src_ref=payload_ref.at[p],
                dst_ref=recv_ref.at[me],
                send_sem=send_sems.at[k - 1],
                recv_sem=recv_sems.at[k - 1],
                device_id=(p,),
                device_id_type=pl.DeviceIdType.MESH,
            )
            rdma.start()
            rdmas.append(rdma)

        local_copy.wait()
        for rdma in rdmas:
            rdma.wait()

    return pl.pallas_call(
        body,
        out_shape=jax.ShapeDtypeStruct((N_DEV, rows, d), jnp.float32),
        in_specs=[pl.BlockSpec(memory_space=pltpu.VMEM)],
        out_specs=pl.BlockSpec(memory_space=pltpu.VMEM),
        scratch_shapes=[
            pltpu.SemaphoreType.DMA,
            pltpu.SemaphoreType.DMA((N_DEV - 1,)),
            pltpu.SemaphoreType.DMA((N_DEV - 1,)),
        ],
        compiler_params=pltpu.CompilerParams(collective_id=0),
    )(payload)


def kernel(x, dest):
    n, d = x.shape
    me = lax.axis_index("i")

    order = jnp.argsort(dest, stable=True)
    xs = x[order]
    counts = jnp.sum(
        dest[None, :] == jnp.arange(N_DEV, dtype=dest.dtype)[:, None], axis=1
    )
    cum = jnp.cumsum(counts)
    offs = cum - counts
    idx = offs[:, None] + jnp.arange(CAP)[None, :]
    valid = jnp.arange(CAP)[None, :] < counts[:, None]
    bucket = jnp.where(
        valid[:, :, None], xs[jnp.clip(idx, 0, n - 1)], 0.0
    )

    dest_rows = lax.bitcast_convert_type(dest, jnp.float32).reshape(-1, d)
    payload = jnp.concatenate(
        [bucket, jnp.broadcast_to(dest_rows[None], (N_DEV,) + dest_rows.shape)],
        axis=1,
    )

    recv = _a2a_pallas(payload)

    dests_all = lax.bitcast_convert_type(
        recv[:, CAP:, :].reshape(N_DEV, -1), jnp.int32
    )
    cin = jnp.sum(dests_all == me, axis=1)
    cum_in = jnp.cumsum(cin)
    offs_in = cum_in - cin
    i = jnp.arange(n)
    src = jnp.searchsorted(cum_in, i, side="right")
    j = i - offs_in[src]
    flat = recv[:, :CAP, :].reshape(N_DEV * CAP, d)
    return flat[src * CAP + j]


# baseline (device time: 43569 ns/iter reference)
import jax
import jax.numpy as jnp
from jax import lax
from jax.experimental import pallas as pl
from jax.experimental.pallas import tpu as pltpu

N_DEV = 4
CAP = 158


def _a2a_pallas(payload):
    _, rows, d = payload.shape

    def body(payload_ref, recv_ref, local_sem, send_sems, recv_sems):
        me = lax.axis_index("i")

        barrier = pltpu.get_barrier_semaphore()
        for k in range(1, N_DEV):
            pl.semaphore_signal(
                barrier,
                inc=1,
                device_id=((me + k) % N_DEV,),
                device_id_type=pl.DeviceIdType.MESH,
            )
        pl.semaphore_wait(barrier, N_DEV - 1)

        local_copy = pltpu.make_async_copy(
            payload_ref.at[me], recv_ref.at[me], local_sem
        )
        local_copy.start()

        rdmas = []
        for k in range(1, N_DEV):
            p = (me + k) % N_DEV
            rdma = pltpu.make_async_remote_copy(
                src_ref=payload_ref.at[p],
                dst_ref=recv_ref.at[me],
                send_sem=send_sems.at[k - 1],
                recv_sem=recv_sems.at[k - 1],
                device_id=(p,),
                device_id_type=pl.DeviceIdType.MESH,
            )
            rdma.start()
            rdmas.append(rdma)

        local_copy.wait()
        for rdma in rdmas:
            rdma.wait()

    return pl.pallas_call(
        body,
        out_shape=jax.ShapeDtypeStruct((N_DEV, rows, d), jnp.float32),
        in_specs=[pl.BlockSpec(memory_space=pltpu.VMEM)],
        out_specs=pl.BlockSpec(memory_space=pltpu.VMEM),
        scratch_shapes=[
            pltpu.SemaphoreType.DMA,
            pltpu.SemaphoreType.DMA((N_DEV - 1,)),
            pltpu.SemaphoreType.DMA((N_DEV - 1,)),
        ],
        compiler_params=pltpu.CompilerParams(collective_id=0),
    )(payload)


def kernel(x, dest):
    n, d = x.shape
    me = lax.axis_index("i")

    order = jnp.argsort(dest, stable=True)
    xs = x[order]
    counts = jnp.sum(
        dest[None, :] == jnp.arange(N_DEV, dtype=dest.dtype)[:, None], axis=1
    )
    cum = jnp.cumsum(counts)
    offs = cum - counts
    idx = offs[:, None] + jnp.arange(CAP)[None, :]
    valid = jnp.arange(CAP)[None, :] < counts[:, None]
    bucket = jnp.where(
        valid[:, :, None], xs[jnp.clip(idx, 0, n - 1)], 0.0
    )

    dest_rows = dest.astype(jnp.float32).reshape(-1, d)
    payload = jnp.concatenate(
        [bucket, jnp.broadcast_to(dest_rows[None], (N_DEV,) + dest_rows.shape)],
        axis=1,
    )

    recv = _a2a_pallas(payload)

    dests_all = recv[:, CAP:, :].reshape(N_DEV, -1).astype(jnp.int32)
    cin = jnp.sum(dests_all == me, axis=1)
    cum_in = jnp.cumsum(cin)
    offs_in = cum_in - cin
    i = jnp.arange(n)
    src = jnp.searchsorted(cum_in, i, side="right")
    j = i - offs_in[src]
    flat = recv[:, :CAP, :].reshape(N_DEV * CAP, d)
    return flat[src * CAP + j]


# device time: 12145 ns/iter; 3.5874x vs baseline; 3.5874x over previous
import jax
import jax.numpy as jnp
from jax import lax
from jax.experimental import pallas as pl
from jax.experimental.pallas import tpu as pltpu

N_DEV = 4
CAP = 158


def kernel(x, dest):
    n, d = x.shape
    rows = CAP + 2

    def body(x_ref, dest_ref, out_ref, pay_ref, recv_ref, local_sem,
             send_sems, recv_sems):
        me = lax.axis_index("i")
        me_f = me.astype(jnp.float32)
        x_val = x_ref[...]
        d_row = dest_ref[...]

        d_bcast = jnp.broadcast_to(d_row, (N_DEV, n))
        r_iota = lax.broadcasted_iota(jnp.int32, (N_DEV, n), 0)
        member = (d_bcast == r_iota).astype(jnp.float32)
        lt = (
            lax.broadcasted_iota(jnp.int32, (n, n), 0)
            < lax.broadcasted_iota(jnp.int32, (n, n), 1)
        ).astype(jnp.float32)
        ranks = lax.dot_general(
            member, lt, (((1,), (0,)), ((), ())),
            preferred_element_type=jnp.float32,
        )

        q_iota = lax.broadcasted_iota(jnp.int32, (CAP, n), 0).astype(jnp.float32)
        d_f = d_row.astype(jnp.float32)
        for r in range(N_DEV):
            p_r = jnp.where(
                ranks[r : r + 1, :] == q_iota, member[r : r + 1, :], 0.0
            )
            pay_ref[r, :CAP, :] = lax.dot_general(
                p_r, x_val, (((1,), (0,)), ((), ())),
                preferred_element_type=jnp.float32,
            )
            pay_ref[r, CAP : CAP + 1, :] = d_f[:, :d]
            pay_ref[r, CAP + 1 :, :] = d_f[:, d:]

        barrier = pltpu.get_barrier_semaphore()
        for k in range(1, N_DEV):
            pl.semaphore_signal(
                barrier, inc=1,
                device_id=((me + k) % N_DEV,),
                device_id_type=pl.DeviceIdType.MESH,
            )
        pl.semaphore_wait(barrier, N_DEV - 1)

        local_copy = pltpu.make_async_copy(
            pay_ref.at[me], recv_ref.at[me], local_sem
        )
        local_copy.start()
        rdmas = []
        for k in range(1, N_DEV):
            p = (me + k) % N_DEV
            rdma = pltpu.make_async_remote_copy(
                src_ref=pay_ref.at[p],
                dst_ref=recv_ref.at[me],
                send_sem=send_sems.at[k - 1],
                recv_sem=recv_sems.at[k - 1],
                device_id=(p,),
                device_id_type=pl.DeviceIdType.MESH,
            )
            rdma.start()
            rdmas.append(rdma)
        local_copy.wait()
        for rdma in rdmas:
            rdma.wait()

        o_iota = lax.broadcasted_iota(jnp.int32, (n, CAP), 0).astype(jnp.float32)
        qo_iota = lax.broadcasted_iota(jnp.int32, (n, CAP), 1).astype(jnp.float32)
        off = jnp.float32(0.0)
        acc = jnp.zeros((n, d), jnp.float32)
        for s in range(N_DEV):
            cin_s = jnp.sum(
                (recv_ref[s, CAP:, :] == me_f).astype(jnp.float32)
            )
            q_s = jnp.where(
                (o_iota == off + qo_iota) & (qo_iota < cin_s), 1.0, 0.0
            )
            acc += lax.dot_general(
                q_s, recv_ref[s, :CAP, :], (((1,), (0,)), ((), ())),
                preferred_element_type=jnp.float32,
            )
            off += cin_s
        out_ref[...] = acc

    return pl.pallas_call(
        body,
        out_shape=jax.ShapeDtypeStruct((n, d), jnp.float32),
        in_specs=[
            pl.BlockSpec(memory_space=pltpu.VMEM),
            pl.BlockSpec(memory_space=pltpu.VMEM),
        ],
        out_specs=pl.BlockSpec(memory_space=pltpu.VMEM),
        scratch_shapes=[
            pltpu.VMEM((N_DEV, rows, d), jnp.float32),
            pltpu.VMEM((N_DEV, rows, d), jnp.float32),
            pltpu.SemaphoreType.DMA,
            pltpu.SemaphoreType.DMA((N_DEV - 1,)),
            pltpu.SemaphoreType.DMA((N_DEV - 1,)),
        ],
        compiler_params=pltpu.CompilerParams(collective_id=0),
    )(x, dest.reshape(1, n))
